# baseline (speedup 1.0000x reference)
import functools

import jax
import jax.numpy as jnp
from jax import lax
from jax.experimental import pallas as pl
from jax.experimental.pallas import tpu as pltpu
from jax.experimental.pallas import tpu_sc as plsc


_TOK_BLK = 512


def _argmin_body(x_ref, cb_ref, idx_ref):
    xb = x_ref[...]
    x_sq = jnp.sum(xb * xb, axis=1)
    n_tok = xb.shape[0]
    k_total = cb_ref.shape[0]
    k_chunk = 128
    acc_min = jnp.full((n_tok,), jnp.inf, dtype=jnp.float32)
    acc_idx = jnp.full((n_tok,), k_total, dtype=jnp.int32)
    for c in range(k_total // k_chunk):
        cbc = cb_ref[pl.ds(c * k_chunk, k_chunk), :]
        cross_t = lax.dot_general(
            cbc, xb, (((1,), (1,)), ((), ())),
            preferred_element_type=jnp.float32)
        c_sq = jnp.sum(cbc * cbc, axis=1)
        d2 = (x_sq[None, :] + c_sq[:, None]) - 2.0 * cross_t
        dist = jnp.sqrt(jnp.clip(d2, 0.0, None))
        cmin = jnp.min(dist, axis=0)
        kiota = lax.broadcasted_iota(jnp.int32, dist.shape, 0) + c * k_chunk
        cidx = jnp.min(
            jnp.where(dist == cmin[None, :], kiota, k_total), axis=0)
        better = cmin < acc_min
        acc_idx = jnp.where(better, cidx, acc_idx)
        acc_min = jnp.where(better, cmin, acc_min)
    idx_ref[0, 0, :] = acc_idx


def _nearest_indices(x2d, codebook):
    n_tok = x2d.shape[0]
    grid = n_tok // _TOK_BLK
    out = pl.pallas_call(
        _argmin_body,
        grid=(grid,),
        in_specs=[
            pl.BlockSpec((_TOK_BLK, x2d.shape[1]), lambda i: (i, 0)),
            pl.BlockSpec(codebook.shape, lambda i: (0, 0)),
        ],
        out_specs=pl.BlockSpec((1, 1, _TOK_BLK), lambda i: (i, 0, 0)),
        out_shape=jax.ShapeDtypeStruct((grid, 1, _TOK_BLK), jnp.int32),
    )(x2d, codebook)
    return out.reshape(n_tok)


def _make_sc_gather(n_rows, d, n_chunks, chunk):
    info = plsc.get_sparse_core_info()
    nc, ns = info.num_cores, info.num_subcores
    nw = nc * ns
    per_w = n_rows // nw
    assert per_w == n_chunks * chunk and chunk <= 128 and chunk % 8 == 0
    mesh = plsc.VectorSubcoreMesh(core_axis_name="c", subcore_axis_name="s")

    @functools.partial(
        pl.kernel, mesh=mesh,
        out_type=jax.ShapeDtypeStruct((n_rows, d), jnp.float32),
        scratch_types=[
            pltpu.VMEM((n_chunks, chunk), jnp.int32),
            pltpu.VMEM((chunk, d), jnp.float32),
            pltpu.SemaphoreType.DMA,
        ],
    )
    def gather(table_hbm, idx_hbm, out_hbm, idx_v, rows_v, sem):
        wid = lax.axis_index("s") * nc + lax.axis_index("c")
        pltpu.sync_copy(idx_hbm.at[wid], idx_v)
        for j in range(n_chunks):
            pltpu.async_copy(table_hbm.at[idx_v.at[j]], rows_v, sem).wait()
            base = (wid * n_chunks + j) * chunk
            pltpu.sync_copy(rows_v, out_hbm.at[pl.ds(base, chunk)])

    return gather


_sc_gather_4608 = _make_sc_gather(4608, 256, n_chunks=2, chunk=72)


def kernel(x, codebook, embedding):
    b, t, d = x.shape
    x2d = x.reshape(b * t, d)
    idx = _nearest_indices(x2d, codebook)
    idx3d = idx.reshape(32, 2, 72)
    out = _sc_gather_4608(embedding, idx3d)
    return out.reshape(b, t, d)


# speedup vs baseline: 29.7811x; 29.7811x over previous
import functools

import jax
import jax.numpy as jnp
from jax import lax
from jax.experimental import pallas as pl
from jax.experimental.pallas import tpu as pltpu
from jax.experimental.pallas import tpu_sc as plsc


_TOK_BLK = 512


def _argmin_body(x_ref, cb_ref, idx_ref):
    xb = x_ref[...]
    cb = cb_ref[...]
    cross = lax.dot_general(
        xb, cb, (((1,), (1,)), ((), ())),
        preferred_element_type=jnp.float32)
    x_sq = jnp.sum(xb * xb, axis=1, keepdims=True)
    c_sq = jnp.sum(cb * cb, axis=1)
    d2 = (x_sq + c_sq[None, :]) - 2.0 * cross
    dist = jnp.sqrt(jnp.clip(d2, 0.0, None))
    dmin = jnp.min(dist, axis=1, keepdims=True)
    k = dist.shape[1]
    kiota = lax.broadcasted_iota(jnp.int32, dist.shape, 1)
    idx = jnp.min(jnp.where(dist == dmin, kiota, k), axis=1)
    idx_ref[0, 0, :] = idx


def _nearest_indices(x2d, codebook):
    n_tok = x2d.shape[0]
    grid = n_tok // _TOK_BLK
    out = pl.pallas_call(
        _argmin_body,
        grid=(grid,),
        in_specs=[
            pl.BlockSpec((_TOK_BLK, x2d.shape[1]), lambda i: (i, 0)),
            pl.BlockSpec(codebook.shape, lambda i: (0, 0)),
        ],
        out_specs=pl.BlockSpec((1, 1, _TOK_BLK), lambda i: (i, 0, 0)),
        out_shape=jax.ShapeDtypeStruct((grid, 1, _TOK_BLK), jnp.int32),
    )(x2d, codebook)
    return out.reshape(n_tok)


def _make_sc_gather(n_rows, d, n_chunks, chunk):
    info = plsc.get_sparse_core_info()
    nc, ns = info.num_cores, info.num_subcores
    nw = nc * ns
    per_w = n_rows // nw
    assert per_w == n_chunks * chunk and chunk <= 128 and chunk % 8 == 0
    mesh = plsc.VectorSubcoreMesh(core_axis_name="c", subcore_axis_name="s")

    @functools.partial(
        pl.kernel, mesh=mesh,
        out_type=jax.ShapeDtypeStruct((n_rows, d), jnp.float32),
        scratch_types=[
            pltpu.VMEM((n_chunks, chunk), jnp.int32),
            pltpu.VMEM((n_chunks, chunk, d), jnp.float32),
            pltpu.SemaphoreType.DMA,
            pltpu.SemaphoreType.DMA,
        ],
    )
    def gather(table_hbm, idx_hbm, out_hbm, idx_v, rows_v, gsem, wsem):
        wid = lax.axis_index("s") * nc + lax.axis_index("c")
        pltpu.sync_copy(idx_hbm.at[wid], idx_v)
        gathers = [
            pltpu.async_copy(table_hbm.at[idx_v.at[j]], rows_v.at[j], gsem)
            for j in range(n_chunks)
        ]
        for g in gathers:
            g.wait()
        writes = [
            pltpu.async_copy(rows_v.at[j],
                             out_hbm.at[pl.ds((wid * n_chunks + j) * chunk,
                                              chunk)], wsem)
            for j in range(n_chunks)
        ]
        for w in writes:
            w.wait()

    return gather


_sc_gather_4608 = _make_sc_gather(4608, 256, n_chunks=2, chunk=72)


def kernel(x, codebook, embedding):
    b, t, d = x.shape
    x2d = x.reshape(b * t, d)
    idx = _nearest_indices(x2d, codebook)
    idx3d = idx.reshape(32, 2, 72)
    out = _sc_gather_4608(embedding, idx3d)
    return out.reshape(b, t, d)


# speedup vs baseline: 31.2174x; 1.0482x over previous
import functools

import jax
import jax.numpy as jnp
from jax import lax
from jax.experimental import pallas as pl
from jax.experimental.pallas import tpu as pltpu
from jax.experimental.pallas import tpu_sc as plsc


_TOK_BLK = 512


def _argmin_body(x_ref, cb_ref, idx_ref):
    xb = x_ref[...]
    cb = cb_ref[...]
    cross = lax.dot_general(
        xb, cb, (((1,), (1,)), ((), ())),
        preferred_element_type=jnp.float32)
    x_sq = jnp.sum(xb * xb, axis=1, keepdims=True)
    c_sq = jnp.sum(cb * cb, axis=1)
    d2 = (x_sq + c_sq[None, :]) - 2.0 * cross
    dist = jnp.sqrt(jnp.clip(d2, 0.0, None))
    dmin = jnp.min(dist, axis=1, keepdims=True)
    k = cross.shape[1]
    kiota = lax.broadcasted_iota(jnp.int32, cross.shape, 1).astype(jnp.float32)
    idx_f = jnp.min(jnp.where(dist == dmin, kiota, float(k)), axis=1)
    idx_ref[0, 0, :] = idx_f.astype(jnp.int32)


def _nearest_indices(x2d, codebook):
    n_tok = x2d.shape[0]
    grid = n_tok // _TOK_BLK
    out = pl.pallas_call(
        _argmin_body,
        grid=(grid,),
        in_specs=[
            pl.BlockSpec((_TOK_BLK, x2d.shape[1]), lambda i: (i, 0)),
            pl.BlockSpec(codebook.shape, lambda i: (0, 0)),
        ],
        out_specs=pl.BlockSpec((1, 1, _TOK_BLK), lambda i: (i, 0, 0)),
        out_shape=jax.ShapeDtypeStruct((grid, 1, _TOK_BLK), jnp.int32),
    )(x2d, codebook)
    return out.reshape(n_tok)


def _make_sc_gather(n_rows, d, n_chunks, chunk):
    info = plsc.get_sparse_core_info()
    nc, ns = info.num_cores, info.num_subcores
    nw = nc * ns
    per_w = n_rows // nw
    assert per_w == n_chunks * chunk and chunk <= 128 and chunk % 8 == 0
    mesh = plsc.VectorSubcoreMesh(core_axis_name="c", subcore_axis_name="s")

    @functools.partial(
        pl.kernel, mesh=mesh,
        out_type=jax.ShapeDtypeStruct((n_rows, d), jnp.float32),
        scratch_types=[
            pltpu.VMEM((n_chunks, chunk), jnp.int32),
            pltpu.VMEM((n_chunks, chunk, d), jnp.float32),
            pltpu.SemaphoreType.DMA,
            pltpu.SemaphoreType.DMA,
        ],
    )
    def gather(table_hbm, idx_hbm, out_hbm, idx_v, rows_v, gsem, wsem):
        wid = lax.axis_index("s") * nc + lax.axis_index("c")
        pltpu.sync_copy(idx_hbm.at[wid], idx_v)
        gathers = [
            pltpu.async_copy(table_hbm.at[idx_v.at[j]], rows_v.at[j], gsem)
            for j in range(n_chunks)
        ]
        for g in gathers:
            g.wait()
        writes = [
            pltpu.async_copy(rows_v.at[j],
                             out_hbm.at[pl.ds((wid * n_chunks + j) * chunk,
                                              chunk)], wsem)
            for j in range(n_chunks)
        ]
        for w in writes:
            w.wait()

    return gather


_sc_gather_4608 = _make_sc_gather(4608, 256, n_chunks=2, chunk=72)


def kernel(x, codebook, embedding):
    b, t, d = x.shape
    x2d = x.reshape(b * t, d)
    idx = _nearest_indices(x2d, codebook)
    idx3d = idx.reshape(32, 2, 72)
    out = _sc_gather_4608(embedding, idx3d)
    return out.reshape(b, t, d)
